# Initial kernel scaffold; baseline (speedup 1.0000x reference)
import jax
import jax.numpy as jnp
from jax.experimental import pallas as pl


def kernel(x, lengths, table, W, b):
    raise NotImplementedError("write your pallas kernel here")


# speedup vs baseline: 2.2946x; 2.2946x over previous
import functools

import jax
import jax.numpy as jnp
from jax import lax
from jax.experimental import pallas as pl
from jax.experimental.pallas import tpu as pltpu
from jax.experimental.pallas import tpu_sc as plsc

BATCH = 4096
SEQ = 200
EMBED = 32
NUM_WORKERS = 32
SAMPLES_PER_WORKER = BATCH // NUM_WORKERS
CHUNK = 16
ROWS_PER_CHUNK = CHUNK * SEQ
GATHER_PIECE = 128
PIECES = ROWS_PER_CHUNK // GATHER_PIECE
NUM_CHUNKS = SAMPLES_PER_WORKER // CHUNK
UNROLL = 8


def _pool_kernel(x_hbm, table_hbm, out_hbm, idx_v, rows_v, out_v, sem):
    cid = lax.axis_index("c")
    sid = lax.axis_index("s")
    wid = sid * 2 + cid
    base = wid * SAMPLES_PER_WORKER

    def chunk_body(g, carry):
        s0 = base + g * CHUNK
        pltpu.sync_copy(x_hbm.at[pl.ds(s0 * SEQ, ROWS_PER_CHUNK)], idx_v)
        copies = [
            pltpu.async_copy(
                table_hbm.at[idx_v.at[pl.ds(j * GATHER_PIECE, GATHER_PIECE)]],
                rows_v.at[pl.ds(j * GATHER_PIECE, GATHER_PIECE)],
                sem,
            )
            for j in range(PIECES)
        ]
        for c in copies:
            c.wait()
        for s in range(CHUNK):
            def rbody(i, acc):
                a0, a1 = acc
                r0 = s * SEQ + i * UNROLL
                for k in range(UNROLL):
                    a0 = a0 + rows_v[r0 + k, pl.ds(0, 16)]
                    a1 = a1 + rows_v[r0 + k, pl.ds(16, 16)]
                return a0, a1

            z = jnp.zeros((16,), jnp.float32)
            a0, a1 = lax.fori_loop(0, SEQ // UNROLL, rbody, (z, z))
            out_v[s, pl.ds(0, 16)] = a0
            out_v[s, pl.ds(16, 16)] = a1
        pltpu.sync_copy(out_v, out_hbm.at[pl.ds(s0, CHUNK)])
        return carry

    lax.fori_loop(0, NUM_CHUNKS, chunk_body, 0)


@functools.partial(jax.jit, static_argnames=())
def _pool(x2, table):
    mesh = plsc.VectorSubcoreMesh(core_axis_name="c", subcore_axis_name="s")
    kern = functools.partial(
        pl.kernel,
        mesh=mesh,
        compiler_params=pltpu.CompilerParams(use_tc_tiling_on_sc=False),
        out_type=jax.ShapeDtypeStruct((BATCH, EMBED), jnp.float32),
        scratch_types=[
            pltpu.VMEM((ROWS_PER_CHUNK,), jnp.int32),
            pltpu.VMEM((ROWS_PER_CHUNK, EMBED), jnp.float32),
            pltpu.VMEM((CHUNK, EMBED), jnp.float32),
            pltpu.SemaphoreType.DMA,
        ],
    )(_pool_kernel)
    return kern(x2, table)


def _head_body(s_ref, l_ref, w_ref, b_ref, o_ref):
    rep = s_ref[...] / l_ref[...]
    rep = jnp.where(rep >= 0, rep, rep * jnp.float32(0.01))
    o_ref[...] = (
        lax.dot_general(
            rep, w_ref[...], (((1,), (1,)), ((), ())),
            preferred_element_type=jnp.float32,
        )
        + b_ref[...]
    )


def _head(sums, lengths_f, W, b2):
    return pl.pallas_call(
        _head_body,
        out_shape=jax.ShapeDtypeStruct((BATCH, 16), jnp.float32),
    )(sums, lengths_f, W, b2)


def kernel(x, lengths, table, W, b):
    x2 = x.reshape(BATCH * SEQ)
    sums = _pool(x2, table)
    lengths_f = lengths.astype(jnp.float32).reshape(BATCH, 1)
    return _head(sums, lengths_f, W, b.reshape(1, 16))
